# baseline (speedup 1.0000x reference)
import functools

import jax
import jax.numpy as jnp
from jax import lax
from jax.experimental import pallas as pl
from jax.experimental.pallas import tpu as pltpu
from jax.experimental.pallas import tpu_sc as plsc

N = 10000
E = 320000
C = 128
NC, NS, L = 2, 16, 16
NW = NC * NS
BR = 16
RING = 4096
WIN = 1024


def _pad_rows(n, ns):
    m = ns * 128
    return ((n + m - 1) // m) * m


def _clamp(rel, own):
    return jnp.minimum(jnp.maximum(rel, 0), own)


def _build_scan_kernel(npad, nblk, cap, own, interpret=False):
    mesh = plsc.VectorSubcoreMesh(core_axis_name="c", subcore_axis_name="s",
                                  num_cores=NC, num_subcores=NS)

    @functools.partial(
        pl.kernel,
        out_type=[
            jax.ShapeDtypeStruct((NW, 1, own), jnp.float32),
            jax.ShapeDtypeStruct((NW, 1, cap), jnp.int32),
            jax.ShapeDtypeStruct((NW, 1, cap), jnp.int32),
            jax.ShapeDtypeStruct((NW, 1, cap), jnp.float32),
            jax.ShapeDtypeStruct((NW, 1, 16), jnp.int32),
        ],
        mesh=mesh,
        interpret=interpret,
        compiler_params=pltpu.CompilerParams(needs_layout_passes=False),
        scratch_types=[
            pltpu.VMEM((2, BR, 128), jnp.int32),
            pltpu.VMEM((2, BR, 128), jnp.int32),
            pltpu.VMEM((2, BR, 128), jnp.float32),
            pltpu.VMEM((RING,), jnp.int32),
            pltpu.VMEM((RING,), jnp.int32),
            pltpu.VMEM((RING,), jnp.float32),
            pltpu.VMEM((own + 16,), jnp.float32),
            pltpu.VMEM((16,), jnp.int32),
            pltpu.SemaphoreType.DMA,
        ],
    )
    def scan_kernel(rowE, colE, ewE, deg_out, lr, lc, lw, cnt,
                    rowb, colb, ewb, stg_r, stg_c, stg_w, dega, cntv, sem):
        cid = lax.axis_index("c")
        sid = lax.axis_index("s")
        wid = cid * NS + sid
        lo = wid * own

        def zs(i, carry):
            stg_r[pl.ds(i * L, L)] = jnp.zeros((L,), jnp.int32)
            stg_c[pl.ds(i * L, L)] = jnp.zeros((L,), jnp.int32)
            stg_w[pl.ds(i * L, L)] = jnp.zeros((L,), jnp.float32)
            return carry
        lax.fori_loop(0, RING // L, zs, 0)

        def zd(i, carry):
            dega[pl.ds(i * L, L)] = jnp.zeros((L,), jnp.float32)
            return carry
        lax.fori_loop(0, (own + 16) // L, zd, 0)

        pltpu.sync_copy(rowE.at[pl.ds(0, BR), :], rowb.at[0])
        pltpu.sync_copy(colE.at[pl.ds(0, BR), :], colb.at[0])
        pltpu.sync_copy(ewE.at[pl.ds(0, BR), :], ewb.at[0])

        def blk_body(blk, carry):
            fillv, flu = carry
            curi = blk & 1
            nxt = 1 - curi
            cp1 = pltpu.async_copy(rowE.at[pl.ds((blk + 1) * BR, BR), :],
                                   rowb.at[nxt], sem)
            cp2 = pltpu.async_copy(colE.at[pl.ds((blk + 1) * BR, BR), :],
                                   colb.at[nxt], sem)
            cp3 = pltpu.async_copy(ewE.at[pl.ds((blk + 1) * BR, BR), :],
                                   ewb.at[nxt], sem)
            for r in range(BR):
                for t in range(128 // L):
                    sl = pl.ds(t * L, L)
                    ic = colb[curi, r, sl]
                    ir = rowb[curi, r, sl]
                    iw = ewb[curi, r, sl]
                    rel = ic - lo
                    owned = (rel >= 0) & (rel < own)
                    relc = jnp.where(owned, rel, jnp.full((L,), own, jnp.int32))
                    plsc.addupdate_scatter(dega, [relc], iw)
                    ones = owned.astype(jnp.int32)
                    pos = (fillv + plsc.cumsum(ones) - 1) & (RING - 1)
                    plsc.store_scatter(stg_r, [pos], ir, mask=owned)
                    plsc.store_scatter(stg_c, [pos], ic, mask=owned)
                    plsc.store_scatter(stg_w, [pos], iw, mask=owned)
                    fillv = fillv + plsc.all_reduce_population_count(owned)
            fil = jnp.max(fillv)
            for _ in range(3):
                can = (fil - flu) >= WIN
                flu_now = flu

                @pl.when(can)
                def _():
                    base = pl.multiple_of(flu_now & (RING - 1), WIN)
                    dst = pl.multiple_of(flu_now, WIN)
                    pltpu.sync_copy(stg_r.at[pl.ds(base, WIN)],
                                    lr.at[wid, 0, pl.ds(dst, WIN)])
                    pltpu.sync_copy(stg_c.at[pl.ds(base, WIN)],
                                    lc.at[wid, 0, pl.ds(dst, WIN)])
                    pltpu.sync_copy(stg_w.at[pl.ds(base, WIN)],
                                    lw.at[wid, 0, pl.ds(dst, WIN)])
                flu = lax.select(can, flu + WIN, flu)
            cp1.wait()
            cp2.wait()
            cp3.wait()
            return fillv, flu

        fillv, flu = lax.fori_loop(
            0, nblk, blk_body,
            (jnp.zeros((L,), jnp.int32), jnp.int32(0)))

        fil = jnp.max(fillv)
        padn = (-fil) & 7
        iot = lax.iota(jnp.int32, L)
        posp = (fillv + iot) & (RING - 1)
        pmask = iot < padn
        plsc.store_scatter(stg_r, [posp], jnp.zeros((L,), jnp.int32),
                           mask=pmask)
        plsc.store_scatter(stg_c, [posp],
                           jnp.full((L,), own, jnp.int32) + lo, mask=pmask)
        plsc.store_scatter(stg_w, [posp], jnp.zeros((L,), jnp.float32),
                           mask=pmask)
        filp = fil + padn
        for _ in range(2):
            can = flu < filp
            flu_now = flu

            @pl.when(can)
            def _():
                base = pl.multiple_of(flu_now & (RING - 1), WIN)
                dst = pl.multiple_of(flu_now, WIN)
                pltpu.sync_copy(stg_r.at[pl.ds(base, WIN)],
                                lr.at[wid, 0, pl.ds(dst, WIN)])
                pltpu.sync_copy(stg_c.at[pl.ds(base, WIN)],
                                lc.at[wid, 0, pl.ds(dst, WIN)])
                pltpu.sync_copy(stg_w.at[pl.ds(base, WIN)],
                                lw.at[wid, 0, pl.ds(dst, WIN)])
            flu = lax.select(can, flu + WIN, flu)

        cntv[pl.ds(0, L)] = jnp.zeros((L,), jnp.int32) + filp
        pltpu.sync_copy(cntv, cnt.at[wid, 0])
        pltpu.sync_copy(dega.at[pl.ds(0, own)], deg_out.at[wid, 0])

    return scan_kernel


def _build_proc_kernel(npad, cap, own, c, interpret=False):
    mesh = plsc.VectorSubcoreMesh(core_axis_name="c", subcore_axis_name="s",
                                  num_cores=NC, num_subcores=NS)
    accr = own + 8

    @functools.partial(
        pl.kernel,
        out_type=jax.ShapeDtypeStruct((NW, own, c), jnp.float32),
        mesh=mesh,
        interpret=interpret,
        compiler_params=pltpu.CompilerParams(needs_layout_passes=False),
        scratch_types=[
            pltpu.VMEM((accr, c), jnp.float32),
            pltpu.VMEM((npad,), jnp.float32),
            pltpu.VMEM((128, c), jnp.float32),
            pltpu.VMEM((128,), jnp.int32),
            pltpu.VMEM((128,), jnp.int32),
            pltpu.VMEM((128,), jnp.float32),
            pltpu.VMEM((16,), jnp.int32),
            pltpu.SemaphoreType.DMA,
        ],
    )
    def proc_kernel(x_hbm, lr, lc, lw, cnt, dinv_hbm, agg_out,
                    acc, dinv_v, rows_v, rr, rc, rw, cntv, sem):
        cid = lax.axis_index("c")
        sid = lax.axis_index("s")
        wid = cid * NS + sid
        lo = wid * own

        def za(i, carry):
            acc[i // (c // L), pl.ds((i % (c // L)) * L, L)] = (
                jnp.zeros((L,), jnp.float32))
            return carry
        lax.fori_loop(0, accr * (c // L), za, 0)

        pltpu.sync_copy(dinv_hbm, dinv_v)
        pltpu.sync_copy(cnt.at[wid, 0], cntv)
        n = jnp.max(cntv[pl.ds(0, L)])
        nb = (n + 127) // 128

        kconsts = [lax.iota(jnp.int32, L) + k * L for k in range(c // L)]

        def batch_body(b, carry):
            base = pl.multiple_of(b * 128, 128)
            pltpu.sync_copy(lr.at[wid, 0, pl.ds(base, 128)], rr)
            pltpu.sync_copy(lc.at[wid, 0, pl.ds(base, 128)], rc)
            pltpu.sync_copy(lw.at[wid, 0, pl.ds(base, 128)], rw)
            cp = pltpu.async_copy(x_hbm.at[rr], rows_v, sem)
            nrs = []
            rels = []
            for t in range(128 // L):
                sl = pl.ds(t * L, L)
                ic = rc[sl]
                iw = rw[sl]
                irow = rr[sl]
                rel = _clamp(ic - lo, own)
                icg = jnp.minimum(ic, npad - 1)
                nr = (plsc.load_gather(dinv_v, [irow]) * iw
                      * plsc.load_gather(dinv_v, [icg]))
                lanepos = base + t * L + lax.iota(jnp.int32, L)
                nr = jnp.where(lanepos < n, nr, 0.0)
                nrs.append(nr)
                rels.append(rel)
            cp.wait()
            for t in range(128 // L):
                for u in range(L):
                    e = t * L + u
                    uu = jnp.full((L,), u, jnp.int32)
                    nsp = nrs[t].at[uu].get(mode="promise_in_bounds")
                    rsp = rels[t].at[uu].get(mode="promise_in_bounds")
                    for k in range(c // L):
                        v = rows_v[e, pl.ds(k * L, L)] * nsp
                        plsc.addupdate_scatter(acc, [rsp, kconsts[k]], v)
            return carry

        lax.fori_loop(0, nb, batch_body, 0)
        pltpu.sync_copy(acc.at[pl.ds(0, own), :], agg_out.at[wid])

    return proc_kernel


def _dinv_body(degp_ref, dinv_ref, d2_ref):
    deg = degp_ref[...] + 1.0
    dinv_ref[...] = lax.rsqrt(deg)
    d2_ref[...] = 1.0 / deg


def _mm_body(a_ref, x_ref, d2_ref, wt_ref, b_ref, o_ref):
    agg = a_ref[...] + x_ref[...] * d2_ref[...]
    o_ref[...] = (
        jnp.dot(agg, wt_ref[...], preferred_element_type=jnp.float32)
        + b_ref[...]
    )


def _run(x, edge_index, edge_weights, W, b, n, e, c, interpret=False):
    npad = _pad_rows(n, NS)
    own = npad // NW
    blk_edges = BR * 128
    nblk = (e + blk_edges - 1) // blk_edges
    epad = (nblk + 1) * blk_edges
    ridx = jnp.pad(edge_index[0], (0, epad - e))
    cidx = jnp.pad(edge_index[1], (0, epad - e))
    ewts = jnp.pad(edge_weights, (0, epad - e))
    rowE = ridx.reshape(epad // 128, 128)
    colE = cidx.reshape(epad // 128, 128)
    ewE = ewts.reshape(epad // 128, 128)
    x_pad = jnp.pad(x, ((0, npad - n), (0, 0)))

    cap = ((nblk * blk_edges + 8192) // WIN) * WIN

    scan_kernel = _build_scan_kernel(npad, nblk, cap, own, interpret)
    proc_kernel = _build_proc_kernel(npad, cap, own, c, interpret)

    deg, lrows, lcols, lwts, cnt = scan_kernel(rowE, colE, ewE)

    rows8 = npad // 128
    dinv2d, d22d = pl.pallas_call(
        _dinv_body,
        out_shape=[jax.ShapeDtypeStruct((rows8, 128), jnp.float32)] * 2,
        interpret=interpret,
    )(deg.reshape(rows8, 128))
    dinv = dinv2d.reshape(npad)
    d2 = d22d.reshape(npad, 1)

    agg_part = proc_kernel(x_pad, lrows, lcols, lwts, cnt, dinv)
    agg = agg_part.reshape(npad, c)

    rb = min(npad, 1024)
    grid = npad // rb
    out_pad = pl.pallas_call(
        _mm_body,
        grid=(grid,),
        in_specs=[
            pl.BlockSpec((rb, c), lambda i: (i, 0)),
            pl.BlockSpec((rb, c), lambda i: (i, 0)),
            pl.BlockSpec((rb, 1), lambda i: (i, 0)),
            pl.BlockSpec((c, c), lambda i: (0, 0)),
            pl.BlockSpec((1, c), lambda i: (0, 0)),
        ],
        out_specs=pl.BlockSpec((rb, c), lambda i: (i, 0)),
        out_shape=jax.ShapeDtypeStruct((npad, c), jnp.float32),
        interpret=interpret,
    )(agg, x_pad, d2, W.T, b.reshape(1, c))
    return out_pad[:n]


def kernel(x, edge_index, edge_weights, W, b):
    return _run(x, edge_index, edge_weights, W, b, N, E, C)


# speedup vs baseline: 1.1691x; 1.1691x over previous
import functools

import jax
import jax.numpy as jnp
from jax import lax
from jax.experimental import pallas as pl
from jax.experimental.pallas import tpu as pltpu
from jax.experimental.pallas import tpu_sc as plsc

N = 10000
E = 320000
C = 128
NC, NS, L = 2, 16, 16
NW = NC * NS
BR = 32
RING = 8192
WIN = 1024


def _pad_rows(n, ns):
    m = ns * 128
    return ((n + m - 1) // m) * m


def _clamp(rel, own):
    return jnp.minimum(jnp.maximum(rel, 0), own)


def _build_scan_kernel(npad, nblk, cap, own, interpret=False):
    mesh = plsc.VectorSubcoreMesh(core_axis_name="c", subcore_axis_name="s",
                                  num_cores=NC, num_subcores=NS)

    @functools.partial(
        pl.kernel,
        out_type=[
            jax.ShapeDtypeStruct((NW, 1, own), jnp.float32),
            jax.ShapeDtypeStruct((NW, 1, cap), jnp.int32),
            jax.ShapeDtypeStruct((NW, 1, cap), jnp.int32),
            jax.ShapeDtypeStruct((NW, 1, cap), jnp.float32),
            jax.ShapeDtypeStruct((NW, 1, 16), jnp.int32),
        ],
        mesh=mesh,
        interpret=interpret,
        compiler_params=pltpu.CompilerParams(needs_layout_passes=False),
        scratch_types=[
            pltpu.VMEM((2, BR, 128), jnp.int32),
            pltpu.VMEM((2, BR, 128), jnp.int32),
            pltpu.VMEM((2, BR, 128), jnp.float32),
            pltpu.VMEM((RING,), jnp.int32),
            pltpu.VMEM((RING,), jnp.int32),
            pltpu.VMEM((RING,), jnp.float32),
            pltpu.VMEM((own + 16,), jnp.float32),
            pltpu.VMEM((16,), jnp.int32),
            pltpu.SemaphoreType.DMA,
        ],
    )
    def scan_kernel(rowE, colE, ewE, deg_out, lr, lc, lw, cnt,
                    rowb, colb, ewb, stg_r, stg_c, stg_w, dega, cntv, sem):
        cid = lax.axis_index("c")
        sid = lax.axis_index("s")
        wid = cid * NS + sid
        lo = wid * own

        def zs(i, carry):
            stg_r[pl.ds(i * L, L)] = jnp.zeros((L,), jnp.int32)
            stg_c[pl.ds(i * L, L)] = jnp.zeros((L,), jnp.int32)
            stg_w[pl.ds(i * L, L)] = jnp.zeros((L,), jnp.float32)
            return carry
        lax.fori_loop(0, RING // L, zs, 0)

        def zd(i, carry):
            dega[pl.ds(i * L, L)] = jnp.zeros((L,), jnp.float32)
            return carry
        lax.fori_loop(0, (own + 16) // L, zd, 0)

        pltpu.sync_copy(rowE.at[pl.ds(0, BR), :], rowb.at[0])
        pltpu.sync_copy(colE.at[pl.ds(0, BR), :], colb.at[0])
        pltpu.sync_copy(ewE.at[pl.ds(0, BR), :], ewb.at[0])

        def blk_body(blk, carry):
            fillv, flu = carry
            curi = blk & 1
            nxt = 1 - curi
            cp1 = pltpu.async_copy(rowE.at[pl.ds((blk + 1) * BR, BR), :],
                                   rowb.at[nxt], sem)
            cp2 = pltpu.async_copy(colE.at[pl.ds((blk + 1) * BR, BR), :],
                                   colb.at[nxt], sem)
            cp3 = pltpu.async_copy(ewE.at[pl.ds((blk + 1) * BR, BR), :],
                                   ewb.at[nxt], sem)
            def row_body(r, fv):
                for t in range(128 // L):
                    sl = pl.ds(t * L, L)
                    ic = colb[curi, r, sl]
                    ir = rowb[curi, r, sl]
                    iw = ewb[curi, r, sl]
                    rel = ic - lo
                    owned = (rel >= 0) & (rel < own)
                    relc = jnp.where(owned, rel,
                                     jnp.full((L,), own, jnp.int32))
                    plsc.addupdate_scatter(dega, [relc], iw)
                    ones = owned.astype(jnp.int32)
                    pos = (fv + plsc.cumsum(ones) - 1) & (RING - 1)
                    plsc.store_scatter(stg_r, [pos], ir, mask=owned)
                    plsc.store_scatter(stg_c, [pos], ic, mask=owned)
                    plsc.store_scatter(stg_w, [pos], iw, mask=owned)
                    fv = fv + plsc.all_reduce_population_count(owned)
                return fv

            fillv = lax.fori_loop(0, BR, row_body, fillv)
            fil = jnp.max(fillv)
            for _ in range(4):
                can = (fil - flu) >= WIN
                flu_now = flu

                @pl.when(can)
                def _():
                    base = pl.multiple_of(flu_now & (RING - 1), WIN)
                    dst = pl.multiple_of(flu_now, WIN)
                    pltpu.sync_copy(stg_r.at[pl.ds(base, WIN)],
                                    lr.at[wid, 0, pl.ds(dst, WIN)])
                    pltpu.sync_copy(stg_c.at[pl.ds(base, WIN)],
                                    lc.at[wid, 0, pl.ds(dst, WIN)])
                    pltpu.sync_copy(stg_w.at[pl.ds(base, WIN)],
                                    lw.at[wid, 0, pl.ds(dst, WIN)])
                flu = lax.select(can, flu + WIN, flu)
            cp1.wait()
            cp2.wait()
            cp3.wait()
            return fillv, flu

        fillv, flu = lax.fori_loop(
            0, nblk, blk_body,
            (jnp.zeros((L,), jnp.int32), jnp.int32(0)))

        fil = jnp.max(fillv)
        padn = (-fil) & 7
        iot = lax.iota(jnp.int32, L)
        posp = (fillv + iot) & (RING - 1)
        pmask = iot < padn
        plsc.store_scatter(stg_r, [posp], jnp.zeros((L,), jnp.int32),
                           mask=pmask)
        plsc.store_scatter(stg_c, [posp],
                           jnp.full((L,), own, jnp.int32) + lo, mask=pmask)
        plsc.store_scatter(stg_w, [posp], jnp.zeros((L,), jnp.float32),
                           mask=pmask)
        filp = fil + padn
        for _ in range(2):
            can = flu < filp
            flu_now = flu

            @pl.when(can)
            def _():
                base = pl.multiple_of(flu_now & (RING - 1), WIN)
                dst = pl.multiple_of(flu_now, WIN)
                pltpu.sync_copy(stg_r.at[pl.ds(base, WIN)],
                                lr.at[wid, 0, pl.ds(dst, WIN)])
                pltpu.sync_copy(stg_c.at[pl.ds(base, WIN)],
                                lc.at[wid, 0, pl.ds(dst, WIN)])
                pltpu.sync_copy(stg_w.at[pl.ds(base, WIN)],
                                lw.at[wid, 0, pl.ds(dst, WIN)])
            flu = lax.select(can, flu + WIN, flu)

        cntv[pl.ds(0, L)] = jnp.zeros((L,), jnp.int32) + filp
        pltpu.sync_copy(cntv, cnt.at[wid, 0])
        pltpu.sync_copy(dega.at[pl.ds(0, own)], deg_out.at[wid, 0])

    return scan_kernel


def _build_proc_kernel(npad, cap, own, c, interpret=False):
    mesh = plsc.VectorSubcoreMesh(core_axis_name="c", subcore_axis_name="s",
                                  num_cores=NC, num_subcores=NS)
    accr = own + 8

    @functools.partial(
        pl.kernel,
        out_type=jax.ShapeDtypeStruct((NW, own, c), jnp.float32),
        mesh=mesh,
        interpret=interpret,
        compiler_params=pltpu.CompilerParams(needs_layout_passes=False),
        scratch_types=[
            pltpu.VMEM((accr, c), jnp.float32),
            pltpu.VMEM((npad,), jnp.float32),
            pltpu.VMEM((2, 256, c), jnp.float32),
            pltpu.VMEM((2, 2, 128), jnp.int32),
            pltpu.VMEM((2, 2, 128), jnp.int32),
            pltpu.VMEM((2, 2, 128), jnp.float32),
            pltpu.VMEM((16,), jnp.int32),
            pltpu.SemaphoreType.DMA,
            pltpu.SemaphoreType.DMA,
        ],
    )
    def proc_kernel(x_hbm, lr, lc, lw, cnt, dinv_hbm, agg_out,
                    acc, dinv_v, rows_v, rr, rc, rw, cntv, semg, semr):
        cid = lax.axis_index("c")
        sid = lax.axis_index("s")
        wid = cid * NS + sid
        lo = wid * own

        def za(i, carry):
            acc[i // (c // L), pl.ds((i % (c // L)) * L, L)] = (
                jnp.zeros((L,), jnp.float32))
            return carry
        lax.fori_loop(0, accr * (c // L), za, 0)

        pltpu.sync_copy(dinv_hbm, dinv_v)
        pltpu.sync_copy(cnt.at[wid, 0], cntv)
        n = jnp.max(cntv[pl.ds(0, L)])
        nb = (n + 255) // 256

        kconsts = [lax.iota(jnp.int32, L) + k * L for k in range(c // L)]

        def load_recs(b, bi, sync):
            base = pl.multiple_of(b * 256, 128)
            cps = []
            for half in range(2):
                off = pl.multiple_of(base + half * 128, 128)
                for src, dst in ((lr, rr), (lc, rc), (lw, rw)):
                    if sync:
                        pltpu.sync_copy(src.at[wid, 0, pl.ds(off, 128)],
                                        dst.at[bi, half])
                    else:
                        cps.append(pltpu.async_copy(
                            src.at[wid, 0, pl.ds(off, 128)],
                            dst.at[bi, half], semr))
            return cps

        def issue_gathers(bi):
            return [
                pltpu.async_copy(x_hbm.at[rr.at[bi, half]],
                                 rows_v.at[bi, pl.ds(half * 128, 128), :],
                                 semg)
                for half in range(2)
            ]

        @pl.when(nb > 0)
        def _():
            load_recs(0, 0, True)
            issue_gathers(0)

        def batch_body(b, carry):
            bi = b & 1
            nbi = 1 - bi
            bn = lax.select(b + 1 < nb, b + 1, 0)
            rec_cps = load_recs(bn, nbi, False)
            pltpu.make_async_copy(
                x_hbm.at[pl.ds(0, 128), :],
                rows_v.at[bi, pl.ds(0, 128), :], semg).wait()
            pltpu.make_async_copy(
                x_hbm.at[pl.ds(0, 128), :],
                rows_v.at[bi, pl.ds(128, 128), :], semg).wait()
            for cp in rec_cps:
                cp.wait()
            issue_gathers(nbi)
            base = b * 256

            def half_body(h, hcarry):
                nrs = []
                rels = []
                for t in range(128 // L):
                    sl = pl.ds(t * L, L)
                    ic = rc[bi, h, sl]
                    iw = rw[bi, h, sl]
                    irow = rr[bi, h, sl]
                    rel = _clamp(ic - lo, own)
                    icg = jnp.minimum(ic, npad - 1)
                    nr = (plsc.load_gather(dinv_v, [irow]) * iw
                          * plsc.load_gather(dinv_v, [icg]))
                    lanepos = (base + h * 128 + t * L
                               + lax.iota(jnp.int32, L))
                    nr = jnp.where(lanepos < n, nr, 0.0)
                    nrs.append(nr)
                    rels.append(rel)
                for t in range(128 // L):
                    for u in range(L):
                        e = h * 128 + t * L + u
                        uu = jnp.full((L,), u, jnp.int32)
                        nsp = nrs[t].at[uu].get(mode="promise_in_bounds")
                        rsp = rels[t].at[uu].get(mode="promise_in_bounds")
                        for k in range(c // L):
                            v = rows_v[bi, e, pl.ds(k * L, L)] * nsp
                            plsc.addupdate_scatter(acc, [rsp, kconsts[k]], v)
                return hcarry

            lax.fori_loop(0, 2, half_body, 0)
            return carry

        lax.fori_loop(0, nb, batch_body, 0)

        @pl.when(nb > 0)
        def _():
            pltpu.make_async_copy(
                x_hbm.at[pl.ds(0, 128), :],
                rows_v.at[0, pl.ds(0, 128), :], semg).wait()
            pltpu.make_async_copy(
                x_hbm.at[pl.ds(0, 128), :],
                rows_v.at[0, pl.ds(128, 128), :], semg).wait()

        pltpu.sync_copy(acc.at[pl.ds(0, own), :], agg_out.at[wid])

    return proc_kernel


def _dinv_body(degp_ref, dinv_ref, d2_ref):
    deg = degp_ref[...] + 1.0
    dinv_ref[...] = lax.rsqrt(deg)
    d2_ref[...] = 1.0 / deg


def _mm_body(a_ref, x_ref, d2_ref, wt_ref, b_ref, o_ref):
    agg = a_ref[...] + x_ref[...] * d2_ref[...]
    o_ref[...] = (
        jnp.dot(agg, wt_ref[...], preferred_element_type=jnp.float32)
        + b_ref[...]
    )


def _run(x, edge_index, edge_weights, W, b, n, e, c, interpret=False):
    npad = _pad_rows(n, NS)
    own = npad // NW
    blk_edges = BR * 128
    nblk = (e + blk_edges - 1) // blk_edges
    epad = (nblk + 1) * blk_edges
    ridx = jnp.pad(edge_index[0], (0, epad - e))
    cidx = jnp.pad(edge_index[1], (0, epad - e))
    ewts = jnp.pad(edge_weights, (0, epad - e))
    rowE = ridx.reshape(epad // 128, 128)
    colE = cidx.reshape(epad // 128, 128)
    ewE = ewts.reshape(epad // 128, 128)
    x_pad = jnp.pad(x, ((0, npad - n), (0, 0)))

    cap = ((nblk * blk_edges + 8192) // WIN) * WIN

    scan_kernel = _build_scan_kernel(npad, nblk, cap, own, interpret)
    proc_kernel = _build_proc_kernel(npad, cap, own, c, interpret)

    deg, lrows, lcols, lwts, cnt = scan_kernel(rowE, colE, ewE)

    rows8 = npad // 128
    dinv2d, d22d = pl.pallas_call(
        _dinv_body,
        out_shape=[jax.ShapeDtypeStruct((rows8, 128), jnp.float32)] * 2,
        interpret=interpret,
    )(deg.reshape(rows8, 128))
    dinv = dinv2d.reshape(npad)
    d2 = d22d.reshape(npad, 1)

    agg_part = proc_kernel(x_pad, lrows, lcols, lwts, cnt, dinv)
    agg = agg_part.reshape(npad, c)

    rb = min(npad, 1024)
    grid = npad // rb
    out_pad = pl.pallas_call(
        _mm_body,
        grid=(grid,),
        in_specs=[
            pl.BlockSpec((rb, c), lambda i: (i, 0)),
            pl.BlockSpec((rb, c), lambda i: (i, 0)),
            pl.BlockSpec((rb, 1), lambda i: (i, 0)),
            pl.BlockSpec((c, c), lambda i: (0, 0)),
            pl.BlockSpec((1, c), lambda i: (0, 0)),
        ],
        out_specs=pl.BlockSpec((rb, c), lambda i: (i, 0)),
        out_shape=jax.ShapeDtypeStruct((npad, c), jnp.float32),
        interpret=interpret,
    )(agg, x_pad, d2, W.T, b.reshape(1, c))
    return out_pad[:n]


def kernel(x, edge_index, edge_weights, W, b):
    return _run(x, edge_index, edge_weights, W, b, N, E, C)
